# baseline (speedup 1.0000x reference)
import functools

import jax
import jax.numpy as jnp
from jax import lax
from jax.experimental import pallas as pl
from jax.experimental.pallas import tpu as pltpu
from jax.experimental.pallas import tpu_sc as plsc

B = 4096
L = 50
HIDDEN = 64
OUT = 128

NC = 2
NS = 16
NW = NC * NS
BPW = B // NW
RPC = 2
TPC = RPC * L
TPCP = 112
NCHUNK = BPW // RPC
NROW2 = B // RPC
VPR = HIDDEN // 16


def _accumulate(rows_ref, par_ref, chunk_row, pooled_ref, out_row):
    cc = [lax.iota(jnp.int32, 16) + 16 * v for v in range(VPR)]
    for r in range(RPC):
        accs = None
        for g in range(L * r // 16, (L * (r + 1) + 15) // 16):
            base_g = par_ref[chunk_row, pl.ds(16 * g, 16)]
            j_lo = max(16 * g, L * r)
            j_hi = min(16 * g + 16, L * (r + 1))
            for j in range(j_lo, j_hi):
                u = j - 16 * g
                pb = base_g.at[jnp.full((16,), u, jnp.int32)].get(
                    mode="promise_in_bounds")
                jfull = jnp.full((16,), j, jnp.int32)
                vals = [
                    plsc.load_gather(rows_ref, [jfull, pb + cc[v]])
                    for v in range(VPR)
                ]
                if accs is None:
                    accs = vals
                else:
                    accs = [a + v_ for a, v_ in zip(accs, vals)]
        for v in range(VPR):
            pooled_ref[out_row + r, pl.ds(16 * v, 16)] = accs[v]


def _pool_body(xp_hbm, par_hbm, table_hbm, out_hbm, idx_v, par_v, rows0,
               rows1, pooled_v, sem0, sem1):
    wid = lax.axis_index("s") * NC + lax.axis_index("c")
    base_irow = wid * NCHUNK

    pltpu.sync_copy(xp_hbm.at[pl.ds(base_irow, NCHUNK)], idx_v)
    pltpu.sync_copy(par_hbm.at[pl.ds(base_irow, NCHUNK)], par_v)

    pltpu.async_copy(table_hbm.at[idx_v.at[0]], rows0, sem0)
    pltpu.async_copy(table_hbm.at[idx_v.at[1]], rows1, sem1)

    def body(i, carry):
        pltpu.make_async_copy(table_hbm.at[idx_v.at[2 * i]], rows0, sem0).wait()
        _accumulate(rows0, par_v, 2 * i, pooled_v, 4 * i)

        @pl.when(i < NCHUNK // 2 - 1)
        def _():
            pltpu.async_copy(table_hbm.at[idx_v.at[2 * i + 2]], rows0, sem0)

        pltpu.make_async_copy(table_hbm.at[idx_v.at[2 * i + 1]], rows1,
                              sem1).wait()
        _accumulate(rows1, par_v, 2 * i + 1, pooled_v, 4 * i + 2)

        @pl.when(i < NCHUNK // 2 - 1)
        def _():
            pltpu.async_copy(table_hbm.at[idx_v.at[2 * i + 3]], rows1, sem1)

        return carry

    lax.fori_loop(0, NCHUNK // 2, body, 0)

    pltpu.sync_copy(pooled_v, out_hbm.at[pl.ds(wid * BPW, BPW)])


_pool = functools.partial(
    pl.kernel,
    out_type=jax.ShapeDtypeStruct((B, HIDDEN), jnp.float32),
    mesh=plsc.VectorSubcoreMesh(core_axis_name="c", subcore_axis_name="s"),
    scratch_types=[
        pltpu.VMEM((NCHUNK, TPCP), jnp.int32),
        pltpu.VMEM((NCHUNK, TPCP), jnp.int32),
        pltpu.VMEM((TPCP, 2 * HIDDEN), jnp.float32),
        pltpu.VMEM((TPCP, 2 * HIDDEN), jnp.float32),
        pltpu.VMEM((BPW, HIDDEN), jnp.float32),
        pltpu.SemaphoreType.DMA,
        pltpu.SemaphoreType.DMA,
    ],
    compiler_params=pltpu.CompilerParams(use_tc_tiling_on_sc=True,
                                         needs_layout_passes=False),
)(_pool_body)


def _mm_body(h_ref, w_ref, b_ref, o_ref):
    o_ref[...] = lax.dot_general(
        h_ref[...], w_ref[...],
        dimension_numbers=(((1,), (1,)), ((), ())),
        preferred_element_type=jnp.float32,
    ) + b_ref[...]


def _linear(h, w, bias):
    blk = 512
    return pl.pallas_call(
        _mm_body,
        grid=(B // blk,),
        in_specs=[
            pl.BlockSpec((blk, HIDDEN), lambda i: (i, 0)),
            pl.BlockSpec((OUT, HIDDEN), lambda i: (0, 0)),
            pl.BlockSpec((1, OUT), lambda i: (0, 0)),
        ],
        out_specs=pl.BlockSpec((blk, OUT), lambda i: (i, 0)),
        out_shape=jax.ShapeDtypeStruct((B, OUT), jnp.float32),
    )(h, w, bias)


def kernel(x, emb_table, W, b):
    t2 = emb_table.reshape(-1, 2 * HIDDEN)
    x3 = x.astype(jnp.int32).reshape(NROW2, TPC)
    pad = jnp.zeros((NROW2, TPCP - TPC), jnp.int32)
    xp = jnp.concatenate([x3 >> 1, pad], axis=1)
    par = jnp.concatenate([(x3 & 1) * HIDDEN, pad], axis=1)
    pooled = _pool(xp, par, t2)
    return _linear(pooled, W, b.reshape(1, OUT))
